# baseline (speedup 1.0000x reference)
import functools
import numpy as np
import jax
import jax.numpy as jnp
from jax import lax
from jax.experimental import pallas as pl
from jax.experimental.pallas import tpu as pltpu
from jax.experimental.pallas import tpu_sc as plsc

N = 10000
E = 320000
D = 128
R = 9
NBASES = 9
H = 8
DH = 16

NC = 2
NS = 16

CHUNKS = 10
CROWS = 9000
ACC_ROWS = 9008
TPT = 568
ESLICE = 2000
EB_A = E // NS
FSTEPS = EB_A // 16
GA = 64

_mesh = plsc.VectorSubcoreMesh(core_axis_name="c", subcore_axis_name="s",
                               num_cores=NC, num_subcores=NS)


@functools.partial(
    pl.kernel,
    out_type=jax.ShapeDtypeStruct((CHUNKS * CROWS, D), jnp.float32),
    mesh=_mesh,
    compiler_params=pltpu.CompilerParams(needs_layout_passes=False),
    scratch_types=[
        pltpu.VMEM_SHARED((ACC_ROWS, D), jnp.float32),
        pltpu.VMEM((ESLICE,), jnp.int32),
        pltpu.VMEM((ESLICE,), jnp.int32),
        pltpu.VMEM((ESLICE,), jnp.int32),
        pltpu.VMEM((EB_A + 2 * GA,), jnp.int32),
        pltpu.VMEM((EB_A + 2 * GA,), jnp.int32),
        pltpu.VMEM((GA, D), jnp.float32),
        pltpu.VMEM((1, GA), jnp.int32),
        pltpu.SemaphoreType.DMA,
    ],
)
def _sc_accumulate(h_hbm, src_hbm, dst_hbm, et_hbm, t_hbm,
                   acc, eb_src, eb_dst, eb_et, sbuf, rbuf, rowbuf,
                   idxst, sem):
    c = lax.axis_index("c")
    sid = lax.axis_index("s")
    base_e = sid * EB_A

    zv = jnp.zeros((16,), jnp.float32)

    zrow0 = jnp.minimum(sid * TPT, ACC_ROWS - TPT)
    wrow = jnp.minimum(sid * TPT, CROWS - TPT)
    for p in range(CHUNKS // NC):
        chunk = NC * p + c
        base_key = chunk * CROWS

        def zrow_body(i, carry):
            for j in range(D // 16):
                rowbuf[i, pl.ds(j * 16, 16)] = zv
            return carry

        lax.fori_loop(0, GA, zrow_body, jnp.int32(0))
        for q in range(8):
            pltpu.sync_copy(rowbuf, acc.at[pl.ds(zrow0 + q * GA, GA)])
        pltpu.sync_copy(rowbuf.at[pl.ds(0, 56)],
                        acc.at[pl.ds(zrow0 + 8 * GA, 56)])
        plsc.subcore_barrier()

        def f_outer(o, cnt):
            sl_h = pl.ds(base_e + o * ESLICE, ESLICE)
            pltpu.sync_copy(src_hbm.at[sl_h], eb_src)
            pltpu.sync_copy(dst_hbm.at[sl_h], eb_dst)
            pltpu.sync_copy(et_hbm.at[sl_h], eb_et)

            def f_body(i, cnt2):
                sl = pl.ds(i * 16, 16)
                d16 = eb_dst[sl]
                t16 = eb_et[sl]
                s16 = eb_src[sl]
                k16 = d16 * R + t16 - base_key
                m = (k16 >= 0) & (k16 < CROWS)
                mi = m.astype(jnp.int32)
                pc = plsc.cumsum(mi)
                lane = lax.iota(jnp.int32, 16)
                pos = jnp.where(m, cnt2 + pc - 1, EB_A + GA + lane)
                plsc.store_scatter(sbuf, [pos], s16)
                plsc.store_scatter(rbuf, [pos], k16)
                return cnt2 + pc[15]

            return lax.fori_loop(0, ESLICE // 16, f_body, cnt)

        cnt = lax.fori_loop(0, EB_A // ESLICE, f_outer, jnp.int32(0))

        pad_s = jnp.zeros((16,), jnp.int32)
        pad_r = CROWS + (lax.iota(jnp.int32, 16) & 7)
        for j in range(GA // 16):
            sbuf[pl.ds(cnt + j * 16, 16)] = pad_s
            rbuf[pl.ds(cnt + j * 16, 16)] = pad_r
        nb = (cnt + (GA - 1)) // GA

        def g_body(b, carry):
            off = b * GA
            for j in range(GA // 16):
                idxst[0, pl.ds(j * 16, 16)] = rbuf[pl.ds(off + j * 16, 16)]
            pltpu.async_copy(h_hbm.at[sbuf.at[pl.ds(off, GA)]], rowbuf,
                             sem).wait()
            pltpu.sync_copy(rowbuf, acc.at[idxst.at[0]], add=True)
            return carry

        lax.fori_loop(0, nb, g_body, jnp.int32(0))
        plsc.subcore_barrier()

        pltpu.sync_copy(acc.at[pl.ds(wrow, TPT)],
                        t_hbm.at[pl.ds(base_key + wrow, TPT)])
        plsc.subcore_barrier()


def _tc_wbuild_body(qb_ref, qc_ref, kb_ref, kc_ref, vb_ref, vc_ref,
                    wq_ref, wk_ref, wv_ref):
    for bref, cref, wref in ((qb_ref, qc_ref, wq_ref),
                             (kb_ref, kc_ref, wk_ref),
                             (vb_ref, vc_ref, wv_ref)):
        w3 = lax.dot_general(cref[...], bref[...],
                             (((1,), (0,)), ((), ())),
                             preferred_element_type=jnp.float32)
        for r in range(R):
            wref[pl.ds(r * D, D), :] = w3[r]


def _tc_wbuild(qb, qc, kb, kc, vb, vc):
    out = jax.ShapeDtypeStruct((R * D, D), jnp.float32)
    return pl.pallas_call(
        _tc_wbuild_body,
        out_shape=(out, out, out),
    )(qb, qc, kb, kc, vb, vc)


_NBLK = 25
_BN = N // _NBLK


def _tc_proj_body(tm_ref, wq_ref, wk_ref, wv_ref, qb_ref, kb_ref, vb_ref,
                  qt_ref, kt_ref, vn_ref):
    t = tm_ref[...]
    qt_ref[...] = jnp.maximum(
        jnp.dot(t, wq_ref[...], preferred_element_type=jnp.float32)
        + qb_ref[...], 0.0)
    kt_ref[...] = jnp.maximum(
        jnp.dot(t, wk_ref[...], preferred_element_type=jnp.float32)
        + kb_ref[...], 0.0)
    vn_ref[...] = jnp.maximum(
        jnp.dot(t, wv_ref[...], preferred_element_type=jnp.float32)
        + vb_ref[...], 0.0)


def _tc_proj(tm, wq, wk, wv, qbias, kbias, vbias):
    out = jax.ShapeDtypeStruct((N, D), jnp.float32)
    wspec = pl.BlockSpec((R * D, D), lambda i: (0, 0))
    bspec = pl.BlockSpec((1, D), lambda i: (0, 0))
    nspec = pl.BlockSpec((_BN, D), lambda i: (i, 0))
    return pl.pallas_call(
        _tc_proj_body,
        grid=(_NBLK,),
        in_specs=[pl.BlockSpec((_BN, R * D), lambda i: (i, 0)),
                  wspec, wspec, wspec, bspec, bspec, bspec],
        out_specs=(nspec, nspec, nspec),
        out_shape=(out, out, out),
    )(tm, wq, wk, wv, qbias, kbias, vbias)


EB_C = 10240
EPAD = EB_C * NC * NS
ESL_C = 2048
BC = 32
ACCW_ROWS = 10008
ACCZ_ROWS = 632
WPT = 632
_INV_SQRT_DH = 0.25


def _lane_perm(x, idx):
    dn = lax.GatherDimensionNumbers(offset_dims=(), collapsed_slice_dims=(0,),
                                    start_index_map=(0,))
    return lax.gather(x, idx.reshape(16, 1), dn, (1,),
                      mode=lax.GatherScatterMode.PROMISE_IN_BOUNDS)


@functools.partial(
    pl.kernel,
    out_type=(jax.ShapeDtypeStruct((NC * N, D), jnp.float32),
              jax.ShapeDtypeStruct((NC * ACCZ_ROWS, D), jnp.float32)),
    mesh=_mesh,
    compiler_params=pltpu.CompilerParams(needs_layout_passes=False),
    scratch_types=[
        pltpu.VMEM_SHARED((ACCW_ROWS, D), jnp.float32),
        pltpu.VMEM_SHARED((ACCZ_ROWS, D), jnp.float32),
        pltpu.VMEM((ESL_C + 16,), jnp.int32),
        pltpu.VMEM((ESL_C + 16,), jnp.int32),
        pltpu.VMEM((BC, D), jnp.float32),
        pltpu.VMEM((BC, D), jnp.float32),
        pltpu.VMEM((BC, D), jnp.float32),
        pltpu.VMEM((BC, D), jnp.float32),
        pltpu.VMEM((BC, D), jnp.float32),
        pltpu.VMEM((BC, D), jnp.float32),
        pltpu.VMEM((BC, D), jnp.float32),
        pltpu.VMEM((BC, D), jnp.float32),
        pltpu.VMEM((1, BC), jnp.int32),
        pltpu.VMEM((1, BC), jnp.int32),
        pltpu.VMEM((1, BC), jnp.int32),
        pltpu.VMEM((1, BC), jnp.int32),
        pltpu.SemaphoreType.DMA,
        pltpu.SemaphoreType.DMA,
        pltpu.SemaphoreType.DMA,
        pltpu.SemaphoreType.DMA,
        pltpu.SemaphoreType.DMA,
        pltpu.SemaphoreType.DMA,
        pltpu.SemaphoreType.DMA,
        pltpu.SemaphoreType.DMA,
        pltpu.SemaphoreType.DMA,
        pltpu.SemaphoreType.DMA,
    ],
)
def _sc_attention(kt_hbm, qt_hbm, vn_hbm, src_hbm, dst_hbm, wv_hbm, z_hbm,
                  accw, accz, ebs, ebd,
                  kba, qba, vba, zba, kbb, qbb, vbb, zbb,
                  ixwa, ixza, ixwb, ixzb,
                  g1a, g2a, g3a, g1b, g2b, g3b, swa, sza, swb, szb):
    c = lax.axis_index("c")
    sid = lax.axis_index("s")
    base_e = jnp.where(c == 0, sid * (4 * ESL_C),
                       NS * 4 * ESL_C + sid * (6 * ESL_C))
    nsl = jnp.where(c == 0, 4, 6)
    zv = jnp.zeros((16,), jnp.float32)
    lane = lax.iota(jnp.int32, 16)
    rot8 = (lane + 8) & 15

    def zfill(i, carry):
        for j in range(D // 16):
            zba[i, pl.ds(j * 16, 16)] = zv
        return carry

    lax.fori_loop(0, BC, zfill, jnp.int32(0))
    woff = jnp.minimum(sid * WPT, ACCW_ROWS - WPT)
    for q in range(19):
        pltpu.sync_copy(zba, accw.at[pl.ds(woff + q * BC, BC)])
    pltpu.sync_copy(zba.at[pl.ds(0, 24)], accw.at[pl.ds(woff + 19 * BC, 24)])
    zoff = jnp.minimum(sid * 40, ACCZ_ROWS - 40)
    pltpu.sync_copy(zba, accz.at[pl.ds(zoff, 32)])
    pltpu.sync_copy(zba.at[pl.ds(0, 8)], accz.at[pl.ds(zoff + 32, 8)])
    plsc.subcore_barrier()

    def eslice(o, carry):
        sl_h = pl.ds(base_e + o * ESL_C, ESL_C)
        pltpu.sync_copy(src_hbm.at[sl_h], ebs.at[pl.ds(0, ESL_C)])
        pltpu.sync_copy(dst_hbm.at[sl_h], ebd.at[pl.ds(0, ESL_C)])
        gsl = base_e + o * ESL_C

        def issue(off, kb_, qb_, vb_, s1, s2, s3):
            sidx = ebs.at[pl.ds(off, BC)]
            didx = ebd.at[pl.ds(off, BC)]
            c1 = pltpu.async_copy(kt_hbm.at[sidx], kb_, s1)
            c2 = pltpu.async_copy(qt_hbm.at[didx], qb_, s2)
            c3 = pltpu.async_copy(vn_hbm.at[sidx], vb_, s3)
            return c1, c2, c3

        def compute(off, kb_, qb_, vb_, zb_, ixw_, ixz_, sw_, sz_):
            def edge_one(i):
                kacc = kb_[i, pl.ds(0, 16)] * qb_[i, pl.ds(0, 16)]
                for j in range(1, D // 16):
                    kacc = kacc + kb_[i, pl.ds(j * 16, 16)] * qb_[i, pl.ds(j * 16, 16)]
                s16 = kacc + _lane_perm(kacc, rot8)
                s16 = jnp.exp(jnp.clip(s16 * _INV_SQRT_DH, -10.0, 10.0))
                for j in range(D // 16):
                    vb_[i, pl.ds(j * 16, 16)] = vb_[i, pl.ds(j * 16, 16)] * s16
                dvec = ebd[pl.ds(off + i, 16)]
                d = dvec[0]
                jv = (d >> 1) & 7
                half = d & 1
                for k in range(D // 16):
                    zb_[i, pl.ds(k * 16, 16)] = zv
                zb_[i, pl.ds(jv * 16, 16)] = jnp.where(
                    (lane >> 3) == half, s16, 0.0)

            def edge(i2, ecarry):
                edge_one(2 * i2)
                edge_one(2 * i2 + 1)
                return ecarry

            lax.fori_loop(0, BC // 2, edge, jnp.int32(0))
            for j4 in range(BC // 16):
                dv = ebd[pl.ds(off + j4 * 16, 16)]
                mreal = (gsl + off + j4 * 16 + lane) < E
                ixw_[0, pl.ds(j4 * 16, 16)] = jnp.where(mreal, dv, N + (lane & 7))
                ixz_[0, pl.ds(j4 * 16, 16)] = jnp.where(mreal, dv >> 4, 625 + (lane & 3))
            cw = pltpu.async_copy(vb_, accw.at[ixw_.at[0]], sw_, add=True)
            cz = pltpu.async_copy(zb_, accz.at[ixz_.at[0]], sz_, add=True)
            return cw, cz

        def pair(p, carry2):
            off0 = (2 * p) * BC
            off1 = off0 + BC
            ga = issue(off0, kba, qba, vba, g1a, g2a, g3a)
            gb = issue(off1, kbb, qbb, vbb, g1b, g2b, g3b)
            for cp in ga:
                cp.wait()
            cwa, cza = compute(off0, kba, qba, vba, zba, ixwa, ixza, swa, sza)
            for cp in gb:
                cp.wait()
            cwb, czb = compute(off1, kbb, qbb, vbb, zbb, ixwb, ixzb, swb, szb)
            cwa.wait()
            cza.wait()
            cwb.wait()
            czb.wait()
            return carry2

        return lax.fori_loop(0, ESL_C // (2 * BC), pair, carry)

    lax.fori_loop(0, nsl, eslice, jnp.int32(0))
    plsc.subcore_barrier()
    row0 = jnp.minimum(sid * WPT, N - WPT)
    pltpu.sync_copy(accw.at[pl.ds(row0, WPT)],
                    wv_hbm.at[pl.ds(c * N + row0, WPT)])
    zw = jnp.minimum(sid * 40, ACCZ_ROWS - 40)
    pltpu.sync_copy(accz.at[pl.ds(zw, 40)],
                    z_hbm.at[pl.ds(c * ACCZ_ROWS + zw, 40)])


def _tc_final_body(wa_ref, wb_ref, za_ref, zb_ref, out_ref):
    wv = wa_ref[0] + wb_ref[0]
    z8 = za_ref[0] + zb_ref[0]
    li = lax.broadcasted_iota(jnp.int32, (H, D), 0)
    ji = lax.broadcasted_iota(jnp.int32, (H, D), 1)
    emat = (li == ji % H).astype(jnp.float32)
    zfull = jnp.dot(z8, emat, preferred_element_type=jnp.float32)
    y = wv / (zfull + 1e-6)
    ai = lax.broadcasted_iota(jnp.int32, (D, D), 0)
    ji2 = lax.broadcasted_iota(jnp.int32, (D, D), 1)
    perm = ((ai % H) * DH + ai // H == ji2).astype(jnp.float32)
    out_ref[...] = jnp.dot(y, perm, preferred_element_type=jnp.float32)


def _tc_final(wvp, zp8):
    return pl.pallas_call(
        _tc_final_body,
        grid=(_NBLK,),
        in_specs=[pl.BlockSpec((1, _BN, D), lambda i: (0, i, 0)),
                  pl.BlockSpec((1, _BN, D), lambda i: (1, i, 0)),
                  pl.BlockSpec((1, _BN, H), lambda i: (0, i, 0)),
                  pl.BlockSpec((1, _BN, H), lambda i: (1, i, 0))],
        out_specs=pl.BlockSpec((_BN, D), lambda i: (i, 0)),
        out_shape=jax.ShapeDtypeStruct((N, D), jnp.float32),
    )(wvp, wvp, zp8, zp8)


def _head_transpose_basis(basis):
    return basis.reshape(NBASES, D, H, DH).transpose(0, 1, 3, 2).reshape(
        NBASES, D, D)


def kernel(h, edge_index, e, q_basis, q_comp, q_bias, k_basis, k_comp, k_bias,
           v_basis, v_comp, v_bias):
    src = edge_index[0]
    dst = edge_index[1]

    qb_t = _head_transpose_basis(q_basis)
    kb_t = _head_transpose_basis(k_basis)
    vb_t = _head_transpose_basis(v_basis)
    qbias_t = q_bias.reshape(H, DH).T.reshape(1, D)
    kbias_t = k_bias.reshape(H, DH).T.reshape(1, D)
    vbias = v_bias.reshape(H, DH).T.reshape(1, D)

    pad = jnp.zeros((EPAD - E,), jnp.int32)
    srcp = jnp.concatenate([src, pad])
    dstp = jnp.concatenate([dst, pad])

    t_acc = _sc_accumulate(h, src, dst, e)
    tm = t_acc.reshape(N, R * D)
    wq, wk, wv = _tc_wbuild(qb_t, q_comp, kb_t, k_comp, vb_t, v_comp)
    qt, kt, vn = _tc_proj(tm, wq, wk, wv, qbias_t, kbias_t, vbias)
    wvp, zp = _sc_attention(kt, qt, vn, srcp, dstp)
    wvp = wvp.reshape(NC, N, D)
    zp8 = zp.reshape(NC, ACCZ_ROWS * D)[:, :N * H].reshape(NC, N, H)
    out = _tc_final(wvp, zp8)
    return out.reshape(N, H, DH)


# speedup vs baseline: 1.1488x; 1.1488x over previous
import functools
import numpy as np
import jax
import jax.numpy as jnp
from jax import lax
from jax.experimental import pallas as pl
from jax.experimental.pallas import tpu as pltpu
from jax.experimental.pallas import tpu_sc as plsc

N = 10000
E = 320000
D = 128
R = 9
NBASES = 9
H = 8
DH = 16

NC = 2
NS = 16

CHUNKS = 10
CROWS = 9000
ACC_ROWS = 9008
TPT = 568
ESLICE = 2000
EB_A = E // NS
FSTEPS = EB_A // 16
GA = 64

_mesh = plsc.VectorSubcoreMesh(core_axis_name="c", subcore_axis_name="s",
                               num_cores=NC, num_subcores=NS)


@functools.partial(
    pl.kernel,
    out_type=jax.ShapeDtypeStruct((CHUNKS * CROWS, D), jnp.float32),
    mesh=_mesh,
    compiler_params=pltpu.CompilerParams(needs_layout_passes=False),
    scratch_types=[
        pltpu.VMEM_SHARED((ACC_ROWS, D), jnp.float32),
        pltpu.VMEM((ESLICE,), jnp.int32),
        pltpu.VMEM((ESLICE,), jnp.int32),
        pltpu.VMEM((ESLICE,), jnp.int32),
        pltpu.VMEM((EB_A + 2 * GA,), jnp.int32),
        pltpu.VMEM((EB_A + 2 * GA,), jnp.int32),
        pltpu.VMEM((GA, D), jnp.float32),
        pltpu.VMEM((1, GA), jnp.int32),
        pltpu.SemaphoreType.DMA,
    ],
)
def _sc_accumulate(h_hbm, src_hbm, dst_hbm, et_hbm, t_hbm,
                   acc, eb_src, eb_dst, eb_et, sbuf, rbuf, rowbuf,
                   idxst, sem):
    c = lax.axis_index("c")
    sid = lax.axis_index("s")
    base_e = sid * EB_A

    zv = jnp.zeros((16,), jnp.float32)

    zrow0 = jnp.minimum(sid * TPT, ACC_ROWS - TPT)
    wrow = jnp.minimum(sid * TPT, CROWS - TPT)
    for p in range(CHUNKS // NC):
        chunk = NC * p + c
        base_key = chunk * CROWS

        def zrow_body(i, carry):
            for j in range(D // 16):
                rowbuf[i, pl.ds(j * 16, 16)] = zv
            return carry

        lax.fori_loop(0, GA, zrow_body, jnp.int32(0))
        for q in range(8):
            pltpu.sync_copy(rowbuf, acc.at[pl.ds(zrow0 + q * GA, GA)])
        pltpu.sync_copy(rowbuf.at[pl.ds(0, 56)],
                        acc.at[pl.ds(zrow0 + 8 * GA, 56)])
        plsc.subcore_barrier()

        def f_outer(o, cnt):
            sl_h = pl.ds(base_e + o * ESLICE, ESLICE)
            pltpu.sync_copy(src_hbm.at[sl_h], eb_src)
            pltpu.sync_copy(dst_hbm.at[sl_h], eb_dst)
            pltpu.sync_copy(et_hbm.at[sl_h], eb_et)

            def f_body(i, cnt2):
                sl = pl.ds(i * 16, 16)
                d16 = eb_dst[sl]
                t16 = eb_et[sl]
                s16 = eb_src[sl]
                k16 = d16 * R + t16 - base_key
                m = (k16 >= 0) & (k16 < CROWS)
                mi = m.astype(jnp.int32)
                pc = plsc.cumsum(mi)
                lane = lax.iota(jnp.int32, 16)
                pos = jnp.where(m, cnt2 + pc - 1, EB_A + GA + lane)
                plsc.store_scatter(sbuf, [pos], s16)
                plsc.store_scatter(rbuf, [pos], k16)
                return cnt2 + pc[15]

            return lax.fori_loop(0, ESLICE // 16, f_body, cnt)

        cnt = lax.fori_loop(0, EB_A // ESLICE, f_outer, jnp.int32(0))

        pad_s = jnp.zeros((16,), jnp.int32)
        pad_r = CROWS + (lax.iota(jnp.int32, 16) & 7)
        for j in range(GA // 16):
            sbuf[pl.ds(cnt + j * 16, 16)] = pad_s
            rbuf[pl.ds(cnt + j * 16, 16)] = pad_r
        nb = (cnt + (GA - 1)) // GA

        def g_body(b, carry):
            off = b * GA
            for j in range(GA // 16):
                idxst[0, pl.ds(j * 16, 16)] = rbuf[pl.ds(off + j * 16, 16)]
            pltpu.async_copy(h_hbm.at[sbuf.at[pl.ds(off, GA)]], rowbuf,
                             sem).wait()
            pltpu.sync_copy(rowbuf, acc.at[idxst.at[0]], add=True)
            return carry

        lax.fori_loop(0, nb, g_body, jnp.int32(0))
        plsc.subcore_barrier()

        pltpu.sync_copy(acc.at[pl.ds(wrow, TPT)],
                        t_hbm.at[pl.ds(base_key + wrow, TPT)])
        plsc.subcore_barrier()


def _tc_wbuild_body(qb_ref, qc_ref, kb_ref, kc_ref, vb_ref, vc_ref,
                    wq_ref, wk_ref, wv_ref):
    for bref, cref, wref in ((qb_ref, qc_ref, wq_ref),
                             (kb_ref, kc_ref, wk_ref),
                             (vb_ref, vc_ref, wv_ref)):
        w3 = lax.dot_general(cref[...], bref[...],
                             (((1,), (0,)), ((), ())),
                             preferred_element_type=jnp.float32)
        for r in range(R):
            wref[pl.ds(r * D, D), :] = w3[r]


def _tc_wbuild(qb, qc, kb, kc, vb, vc):
    out = jax.ShapeDtypeStruct((R * D, D), jnp.float32)
    return pl.pallas_call(
        _tc_wbuild_body,
        out_shape=(out, out, out),
    )(qb, qc, kb, kc, vb, vc)


_NBLK = 25
_BN = N // _NBLK


def _tc_proj_body(tm_ref, wq_ref, wk_ref, wv_ref, qb_ref, kb_ref, vb_ref,
                  qt_ref, kt_ref, vn_ref):
    t = tm_ref[...]
    qt_ref[...] = jnp.maximum(
        jnp.dot(t, wq_ref[...], preferred_element_type=jnp.float32)
        + qb_ref[...], 0.0)
    kt_ref[...] = jnp.maximum(
        jnp.dot(t, wk_ref[...], preferred_element_type=jnp.float32)
        + kb_ref[...], 0.0)
    vn_ref[...] = jnp.maximum(
        jnp.dot(t, wv_ref[...], preferred_element_type=jnp.float32)
        + vb_ref[...], 0.0)


def _tc_proj(tm, wq, wk, wv, qbias, kbias, vbias):
    out = jax.ShapeDtypeStruct((N, D), jnp.float32)
    wspec = pl.BlockSpec((R * D, D), lambda i: (0, 0))
    bspec = pl.BlockSpec((1, D), lambda i: (0, 0))
    nspec = pl.BlockSpec((_BN, D), lambda i: (i, 0))
    return pl.pallas_call(
        _tc_proj_body,
        grid=(_NBLK,),
        in_specs=[pl.BlockSpec((_BN, R * D), lambda i: (i, 0)),
                  wspec, wspec, wspec, bspec, bspec, bspec],
        out_specs=(nspec, nspec, nspec),
        out_shape=(out, out, out),
    )(tm, wq, wk, wv, qbias, kbias, vbias)


EB_C = 10240
EPAD = EB_C * NC * NS
ESL_C = 2048
BC = 32
ACCW_ROWS = 10008
ACCZ_ROWS = 632
WPT = 632
_INV_SQRT_DH = 0.25


def _lane_perm(x, idx):
    dn = lax.GatherDimensionNumbers(offset_dims=(), collapsed_slice_dims=(0,),
                                    start_index_map=(0,))
    return lax.gather(x, idx.reshape(16, 1), dn, (1,),
                      mode=lax.GatherScatterMode.PROMISE_IN_BOUNDS)


@functools.partial(
    pl.kernel,
    out_type=(jax.ShapeDtypeStruct((NC * N, D), jnp.float32),
              jax.ShapeDtypeStruct((NC * ACCZ_ROWS, D), jnp.float32)),
    mesh=_mesh,
    compiler_params=pltpu.CompilerParams(needs_layout_passes=False),
    scratch_types=[
        pltpu.VMEM_SHARED((ACCW_ROWS, D), jnp.float32),
        pltpu.VMEM_SHARED((ACCZ_ROWS, D), jnp.float32),
        pltpu.VMEM((ESL_C + 16,), jnp.int32),
        pltpu.VMEM((ESL_C + 16,), jnp.int32),
        pltpu.VMEM((BC, D), jnp.float32),
        pltpu.VMEM((BC, D), jnp.float32),
        pltpu.VMEM((BC, D), jnp.float32),
        pltpu.VMEM((BC, D), jnp.float32),
        pltpu.VMEM((BC, D), jnp.float32),
        pltpu.VMEM((BC, D), jnp.float32),
        pltpu.VMEM((BC, D), jnp.float32),
        pltpu.VMEM((BC, D), jnp.float32),
        pltpu.VMEM((1, BC), jnp.int32),
        pltpu.VMEM((1, BC), jnp.int32),
        pltpu.VMEM((1, BC), jnp.int32),
        pltpu.VMEM((1, BC), jnp.int32),
        pltpu.SemaphoreType.DMA,
        pltpu.SemaphoreType.DMA,
        pltpu.SemaphoreType.DMA,
        pltpu.SemaphoreType.DMA,
        pltpu.SemaphoreType.DMA,
        pltpu.SemaphoreType.DMA,
        pltpu.SemaphoreType.DMA,
        pltpu.SemaphoreType.DMA,
        pltpu.SemaphoreType.DMA,
        pltpu.SemaphoreType.DMA,
    ],
)
def _sc_attention(kt_hbm, qt_hbm, vn_hbm, src_hbm, dst_hbm, wv_hbm, z_hbm,
                  accw, accz, ebs, ebd,
                  kba, qba, vba, zba, kbb, qbb, vbb, zbb,
                  ixwa, ixza, ixwb, ixzb,
                  g1a, g2a, g3a, g1b, g2b, g3b, swa, sza, swb, szb):
    c = lax.axis_index("c")
    sid = lax.axis_index("s")
    base_e = jnp.where(c == 0, sid * (6 * ESL_C),
                       NS * 6 * ESL_C + sid * (4 * ESL_C))
    nsl = jnp.where(c == 0, 6, 4)
    zv = jnp.zeros((16,), jnp.float32)
    lane = lax.iota(jnp.int32, 16)
    rot8 = (lane + 8) & 15

    def zfill(i, carry):
        for j in range(D // 16):
            zba[i, pl.ds(j * 16, 16)] = zv
        return carry

    lax.fori_loop(0, BC, zfill, jnp.int32(0))
    woff = jnp.minimum(sid * WPT, ACCW_ROWS - WPT)
    for q in range(19):
        pltpu.sync_copy(zba, accw.at[pl.ds(woff + q * BC, BC)])
    pltpu.sync_copy(zba.at[pl.ds(0, 24)], accw.at[pl.ds(woff + 19 * BC, 24)])
    zoff = jnp.minimum(sid * 40, ACCZ_ROWS - 40)
    pltpu.sync_copy(zba, accz.at[pl.ds(zoff, 32)])
    pltpu.sync_copy(zba.at[pl.ds(0, 8)], accz.at[pl.ds(zoff + 32, 8)])
    plsc.subcore_barrier()

    def eslice(o, carry):
        sl_h = pl.ds(base_e + o * ESL_C, ESL_C)
        pltpu.sync_copy(src_hbm.at[sl_h], ebs.at[pl.ds(0, ESL_C)])
        pltpu.sync_copy(dst_hbm.at[sl_h], ebd.at[pl.ds(0, ESL_C)])
        gsl = base_e + o * ESL_C

        def issue(off, kb_, qb_, vb_, s1, s2, s3):
            sidx = ebs.at[pl.ds(off, BC)]
            didx = ebd.at[pl.ds(off, BC)]
            c1 = pltpu.async_copy(kt_hbm.at[sidx], kb_, s1)
            c2 = pltpu.async_copy(qt_hbm.at[didx], qb_, s2)
            c3 = pltpu.async_copy(vn_hbm.at[sidx], vb_, s3)
            return c1, c2, c3

        def compute(off, kb_, qb_, vb_, zb_, ixw_, ixz_, sw_, sz_):
            def edge_one(i):
                kacc = kb_[i, pl.ds(0, 16)] * qb_[i, pl.ds(0, 16)]
                for j in range(1, D // 16):
                    kacc = kacc + kb_[i, pl.ds(j * 16, 16)] * qb_[i, pl.ds(j * 16, 16)]
                s16 = kacc + _lane_perm(kacc, rot8)
                s16 = jnp.exp(jnp.clip(s16 * _INV_SQRT_DH, -10.0, 10.0))
                for j in range(D // 16):
                    vb_[i, pl.ds(j * 16, 16)] = vb_[i, pl.ds(j * 16, 16)] * s16
                dvec = ebd[pl.ds(off + i, 16)]
                d = dvec[0]
                jv = (d >> 1) & 7
                half = d & 1
                for k in range(D // 16):
                    zb_[i, pl.ds(k * 16, 16)] = zv
                zb_[i, pl.ds(jv * 16, 16)] = jnp.where(
                    (lane >> 3) == half, s16, 0.0)

            def edge(i2, ecarry):
                edge_one(2 * i2)
                edge_one(2 * i2 + 1)
                return ecarry

            lax.fori_loop(0, BC // 2, edge, jnp.int32(0))
            for j4 in range(BC // 16):
                dv = ebd[pl.ds(off + j4 * 16, 16)]
                mreal = (gsl + off + j4 * 16 + lane) < E
                ixw_[0, pl.ds(j4 * 16, 16)] = jnp.where(mreal, dv, N + (lane & 7))
                ixz_[0, pl.ds(j4 * 16, 16)] = jnp.where(mreal, dv >> 4, 625 + (lane & 3))
            cw = pltpu.async_copy(vb_, accw.at[ixw_.at[0]], sw_, add=True)
            cz = pltpu.async_copy(zb_, accz.at[ixz_.at[0]], sz_, add=True)
            return cw, cz

        def pair(p, carry2):
            off0 = (2 * p) * BC
            off1 = off0 + BC
            ga = issue(off0, kba, qba, vba, g1a, g2a, g3a)
            gb = issue(off1, kbb, qbb, vbb, g1b, g2b, g3b)
            for cp in ga:
                cp.wait()
            cwa, cza = compute(off0, kba, qba, vba, zba, ixwa, ixza, swa, sza)
            for cp in gb:
                cp.wait()
            cwb, czb = compute(off1, kbb, qbb, vbb, zbb, ixwb, ixzb, swb, szb)
            cwa.wait()
            cza.wait()
            cwb.wait()
            czb.wait()
            return carry2

        return lax.fori_loop(0, ESL_C // (2 * BC), pair, carry)

    lax.fori_loop(0, nsl, eslice, jnp.int32(0))
    plsc.subcore_barrier()
    row0 = jnp.minimum(sid * WPT, N - WPT)
    pltpu.sync_copy(accw.at[pl.ds(row0, WPT)],
                    wv_hbm.at[pl.ds(c * N + row0, WPT)])
    zw = jnp.minimum(sid * 40, ACCZ_ROWS - 40)
    pltpu.sync_copy(accz.at[pl.ds(zw, 40)],
                    z_hbm.at[pl.ds(c * ACCZ_ROWS + zw, 40)])


def _tc_final_body(wa_ref, wb_ref, za_ref, zb_ref, out_ref):
    wv = wa_ref[0] + wb_ref[0]
    z8 = za_ref[0] + zb_ref[0]
    li = lax.broadcasted_iota(jnp.int32, (H, D), 0)
    ji = lax.broadcasted_iota(jnp.int32, (H, D), 1)
    emat = (li == ji % H).astype(jnp.float32)
    zfull = jnp.dot(z8, emat, preferred_element_type=jnp.float32)
    y = wv / (zfull + 1e-6)
    ai = lax.broadcasted_iota(jnp.int32, (D, D), 0)
    ji2 = lax.broadcasted_iota(jnp.int32, (D, D), 1)
    perm = ((ai % H) * DH + ai // H == ji2).astype(jnp.float32)
    out_ref[...] = jnp.dot(y, perm, preferred_element_type=jnp.float32)


def _tc_final(wvp, zp8):
    return pl.pallas_call(
        _tc_final_body,
        grid=(_NBLK,),
        in_specs=[pl.BlockSpec((1, _BN, D), lambda i: (0, i, 0)),
                  pl.BlockSpec((1, _BN, D), lambda i: (1, i, 0)),
                  pl.BlockSpec((1, _BN, H), lambda i: (0, i, 0)),
                  pl.BlockSpec((1, _BN, H), lambda i: (1, i, 0))],
        out_specs=pl.BlockSpec((_BN, D), lambda i: (i, 0)),
        out_shape=jax.ShapeDtypeStruct((N, D), jnp.float32),
    )(wvp, wvp, zp8, zp8)


def _head_transpose_basis(basis):
    return basis.reshape(NBASES, D, H, DH).transpose(0, 1, 3, 2).reshape(
        NBASES, D, D)


def kernel(h, edge_index, e, q_basis, q_comp, q_bias, k_basis, k_comp, k_bias,
           v_basis, v_comp, v_bias):
    src = edge_index[0]
    dst = edge_index[1]

    qb_t = _head_transpose_basis(q_basis)
    kb_t = _head_transpose_basis(k_basis)
    vb_t = _head_transpose_basis(v_basis)
    qbias_t = q_bias.reshape(H, DH).T.reshape(1, D)
    kbias_t = k_bias.reshape(H, DH).T.reshape(1, D)
    vbias = v_bias.reshape(H, DH).T.reshape(1, D)

    pad = jnp.zeros((EPAD - E,), jnp.int32)
    srcp = jnp.concatenate([src, pad])
    dstp = jnp.concatenate([dst, pad])

    t_acc = _sc_accumulate(h, src, dst, e)
    tm = t_acc.reshape(N, R * D)
    wq, wk, wv = _tc_wbuild(qb_t, q_comp, kb_t, k_comp, vb_t, v_comp)
    qt, kt, vn = _tc_proj(tm, wq, wk, wv, qbias_t, kbias_t, vbias)
    wvp, zp = _sc_attention(kt, qt, vn, srcp, dstp)
    wvp = wvp.reshape(NC, N, D)
    zp8 = zp.reshape(NC, ACCZ_ROWS * D)[:, :N * H].reshape(NC, N, H)
    out = _tc_final(wvp, zp8)
    return out.reshape(N, H, DH)
